# baseline (speedup 1.0000x reference)
import jax
import jax.numpy as jnp
from jax import lax
from jax.experimental import pallas as pl
from jax.experimental.pallas import tpu as pltpu
from jax.experimental.pallas import tpu_sc as plsc

N = 10000
D = 128
H = 1024
NCH = H // 128
BN = 400
NW = 32
EB = 128
R_MAIN = 632
NB = 80
R_LAST = N - 15 * R_MAIN
AGG_ROWS = N + 8


def _pre_body(x_ref, wpre_ref, bpre_ref, wmp_ref, bmp_ref, h_ref, *z_refs):
    xb = x_ref[...]
    h = jnp.tanh(jnp.dot(xb, wpre_ref[...], preferred_element_type=jnp.float32)
                 + bpre_ref[...])
    z = jnp.dot(h, wmp_ref[...], preferred_element_type=jnp.float32) + bmp_ref[...]
    h_ref[...] = h
    for c in range(NCH):
        z_refs[c][...] = z[:, c * 128:(c + 1) * 128]


def _pre(x, W_pre, b_pre, W_mp, b_mp):
    grid = (N // BN,)
    return pl.pallas_call(
        _pre_body,
        grid=grid,
        in_specs=[
            pl.BlockSpec((BN, D), lambda i: (i, 0)),
            pl.BlockSpec((D, H), lambda i: (0, 0)),
            pl.BlockSpec((1, H), lambda i: (0, 0)),
            pl.BlockSpec((H, H), lambda i: (0, 0)),
            pl.BlockSpec((1, H), lambda i: (0, 0)),
        ],
        out_specs=[pl.BlockSpec((BN, H), lambda i: (i, 0))]
        + [pl.BlockSpec((BN, 128), lambda i: (i, 0)) for _ in range(NCH)],
        out_shape=[jax.ShapeDtypeStruct((N, H), jnp.float32)]
        + [jax.ShapeDtypeStruct((N, 128), jnp.float32) for _ in range(NCH)],
    )(x, W_pre, b_pre.reshape(1, H), W_mp, b_mp.reshape(1, H))


def _segsum_body(src_hbm, dst_hbm, *rest):
    z_hbms = rest[:NCH]
    out_hbm = rest[NCH]
    sidx_all, didx_all, rows_a, agg_sh, sem_a = rest[NCH + 1:]

    core = lax.axis_index("c")
    sub = lax.axis_index("s")
    wid = core * 16 + sub
    r0 = sub * R_MAIN
    last = sub == 15
    zero16 = jnp.zeros((16,), jnp.float32)

    def _zero_rows_a():
        def _zrow(r, _):
            for k in range(128 // 16):
                rows_a[r, pl.ds(k * 16, 16)] = zero16
            return _

        lax.fori_loop(0, EB, _zrow, None)

    def _zero_span(base, total):
        for off in range(0, total, EB):
            n = min(EB, total - off)
            pltpu.sync_copy(rows_a.at[pl.ds(0, n)],
                            agg_sh.at[pl.ds(base + off, n)])

    def _zero_my_slice():
        _zero_rows_a()

        @pl.when(jnp.logical_not(last))
        def _():
            _zero_span(r0, R_MAIN)

        @pl.when(last)
        def _():
            _zero_span(r0, R_LAST + 8)

    _zero_my_slice()
    plsc.subcore_barrier()

    nb = src_hbm.shape[1]
    for c in range(NCH):
        z_hbm = z_hbms[c]

        def _batch(b, _):
            pltpu.sync_copy(src_hbm.at[wid, b], sidx_all)
            pltpu.sync_copy(dst_hbm.at[wid, b], didx_all)
            pltpu.async_copy(z_hbm.at[sidx_all], rows_a, sem_a).wait()
            pltpu.sync_copy(rows_a, agg_sh.at[didx_all], add=True)
            return _

        lax.fori_loop(0, nb, _batch, None)
        plsc.subcore_barrier()

        @pl.when(jnp.logical_not(last))
        def _():
            pltpu.sync_copy(agg_sh.at[pl.ds(r0, R_MAIN)],
                            out_hbm.at[core, c, pl.ds(r0, R_MAIN)])

        @pl.when(last)
        def _():
            pltpu.sync_copy(agg_sh.at[pl.ds(r0, R_LAST)],
                            out_hbm.at[core, c, pl.ds(r0, R_LAST)])

        if c + 1 < NCH:
            _zero_my_slice()
            plsc.subcore_barrier()


def _segsum(src_r, dst_r, zs):
    mesh = plsc.VectorSubcoreMesh(core_axis_name="c", subcore_axis_name="s")
    f = pl.kernel(
        _segsum_body,
        mesh=mesh,
        out_type=jax.ShapeDtypeStruct((2, NCH, N, 128), jnp.float32),
        scratch_types=[
            pltpu.VMEM((EB,), jnp.int32),
            pltpu.VMEM((EB,), jnp.int32),
            pltpu.VMEM((EB, 128), jnp.float32),
            pltpu.VMEM_SHARED((AGG_ROWS, 128), jnp.float32),
            pltpu.SemaphoreType.DMA,
        ],
    )
    return f(src_r, dst_r, *zs)


def _post_body(agg_ref, h_ref, wpost_ref, bpost_ref, o_ref):
    acc = bpost_ref[...] + jnp.dot(
        h_ref[...], wpost_ref[H:, :], preferred_element_type=jnp.float32)
    for c in range(NCH):
        g = jnp.tanh(agg_ref[0, c] + agg_ref[1, c])
        acc += jnp.dot(g, wpost_ref[c * 128:(c + 1) * 128, :],
                       preferred_element_type=jnp.float32)
    o_ref[...] = jnp.maximum(acc, 0.0) + jnp.log1p(jnp.exp(-jnp.abs(acc)))


def _post(agg, h, W_post, b_post):
    grid = (N // BN,)
    return pl.pallas_call(
        _post_body,
        grid=grid,
        in_specs=[
            pl.BlockSpec((2, NCH, BN, 128), lambda i: (0, 0, i, 0)),
            pl.BlockSpec((BN, H), lambda i: (i, 0)),
            pl.BlockSpec((2 * H, D), lambda i: (0, 0)),
            pl.BlockSpec((1, D), lambda i: (0, 0)),
        ],
        out_specs=pl.BlockSpec((BN, D), lambda i: (i, 0)),
        out_shape=jax.ShapeDtypeStruct((N, D), jnp.float32),
    )(agg, h, W_post, b_post.reshape(1, D))


def kernel(x, edge_index, W_pre, b_pre, W_mp, b_mp, W_post, b_post):
    src = edge_index[0].astype(jnp.int32)
    dst = edge_index[1].astype(jnp.int32)
    e = src.shape[0]
    nb = NB
    pad = NW * EB * nb - e
    src_r = jnp.concatenate([src, jnp.zeros((pad,), jnp.int32)]).reshape(NW, nb, EB)
    dst_r = jnp.concatenate([dst, jnp.full((pad,), N, jnp.int32)]).reshape(NW, nb, EB)

    h, *zs = _pre(x, W_pre, b_pre, W_mp, b_mp)
    agg = _segsum(src_r, dst_r, zs)
    return _post(agg, h, W_post, b_post)


# speedup vs baseline: 2.3713x; 2.3713x over previous
import jax
import jax.numpy as jnp
from jax import lax
from jax.experimental import pallas as pl
from jax.experimental.pallas import tpu as pltpu
from jax.experimental.pallas import tpu_sc as plsc

N = 10000
D = 128
H = 1024
NCH = H // 128
BN = 400
NW = 32
EB = 128
R_MAIN = 632
E_TOTAL = 320000
NB = -(-E_TOTAL // (NW * EB))
_EL = E_TOTAL - (NW - 1) * NB * EB
NB_LAST = -(-_EL // EB)
R_LAST = N - 15 * R_MAIN
AGG_ROWS = N + 8


def _pre_body(x_ref, wpre_ref, bpre_ref, wmp_ref, bmp_ref, h_ref, *z_refs):
    xb = x_ref[...]
    h = jnp.tanh(jnp.dot(xb, wpre_ref[...], preferred_element_type=jnp.float32)
                 + bpre_ref[...])
    z = jnp.dot(h, wmp_ref[...], preferred_element_type=jnp.float32) + bmp_ref[...]
    h_ref[...] = h
    for c in range(NCH):
        z_refs[c][...] = z[:, c * 128:(c + 1) * 128]


def _pre(x, W_pre, b_pre, W_mp, b_mp):
    grid = (N // BN,)
    return pl.pallas_call(
        _pre_body,
        grid=grid,
        in_specs=[
            pl.BlockSpec((BN, D), lambda i: (i, 0)),
            pl.BlockSpec((D, H), lambda i: (0, 0)),
            pl.BlockSpec((1, H), lambda i: (0, 0)),
            pl.BlockSpec((H, H), lambda i: (0, 0)),
            pl.BlockSpec((1, H), lambda i: (0, 0)),
        ],
        out_specs=[pl.BlockSpec((BN, H), lambda i: (i, 0))]
        + [pl.BlockSpec((BN, 128), lambda i: (i, 0)) for _ in range(NCH)],
        out_shape=[jax.ShapeDtypeStruct((N, H), jnp.float32)]
        + [jax.ShapeDtypeStruct((N, 128), jnp.float32) for _ in range(NCH)],
    )(x, W_pre, b_pre.reshape(1, H), W_mp, b_mp.reshape(1, H))


def _segsum_body(src_hbm, dst_hbm, *rest):
    z_hbms = rest[:NCH]
    out_hbm = rest[NCH]
    sidx_all, didx_all, rows_a, agg_sh, sem_a = rest[NCH + 1:]

    core = lax.axis_index("c")
    sub = lax.axis_index("s")
    wid = core * 16 + sub
    r0 = sub * R_MAIN
    last = sub == 15
    zero16 = jnp.zeros((16,), jnp.float32)

    def _zero_rows_a():
        def _zrow(r, _):
            for k in range(128 // 16):
                rows_a[r, pl.ds(k * 16, 16)] = zero16
            return _

        lax.fori_loop(0, EB, _zrow, None)

    def _zero_span(base, total):
        for off in range(0, total, EB):
            n = min(EB, total - off)
            pltpu.sync_copy(rows_a.at[pl.ds(0, n)],
                            agg_sh.at[pl.ds(base + off, n)])

    def _zero_my_slice():
        _zero_rows_a()

        @pl.when(jnp.logical_not(last))
        def _():
            _zero_span(r0, R_MAIN)

        @pl.when(last)
        def _():
            _zero_span(r0, R_LAST + 8)

    _zero_my_slice()
    plsc.subcore_barrier()

    nbw = jnp.where(wid == NW - 1, NB_LAST, NB)
    for c in range(NCH):
        z_hbm = z_hbms[c]

        def _batch(b, _):
            pltpu.sync_copy(src_hbm.at[wid, b], sidx_all)
            pltpu.sync_copy(dst_hbm.at[wid, b], didx_all)
            pltpu.async_copy(z_hbm.at[sidx_all], rows_a, sem_a).wait()
            pltpu.sync_copy(rows_a, agg_sh.at[didx_all], add=True)
            return _

        lax.fori_loop(0, nbw, _batch, None)
        plsc.subcore_barrier()

        @pl.when(jnp.logical_not(last))
        def _():
            pltpu.sync_copy(agg_sh.at[pl.ds(r0, R_MAIN)],
                            out_hbm.at[core, c, pl.ds(r0, R_MAIN)])

        @pl.when(last)
        def _():
            pltpu.sync_copy(agg_sh.at[pl.ds(r0, R_LAST)],
                            out_hbm.at[core, c, pl.ds(r0, R_LAST)])

        if c + 1 < NCH:
            _zero_my_slice()
            plsc.subcore_barrier()


def _segsum(src_r, dst_r, zs):
    mesh = plsc.VectorSubcoreMesh(core_axis_name="c", subcore_axis_name="s")
    f = pl.kernel(
        _segsum_body,
        mesh=mesh,
        out_type=jax.ShapeDtypeStruct((2, NCH, N, 128), jnp.float32),
        scratch_types=[
            pltpu.VMEM((EB,), jnp.int32),
            pltpu.VMEM((EB,), jnp.int32),
            pltpu.VMEM((EB, 128), jnp.float32),
            pltpu.VMEM_SHARED((AGG_ROWS, 128), jnp.float32),
            pltpu.SemaphoreType.DMA,
        ],
    )
    return f(src_r, dst_r, *zs)


def _post_body(agg_ref, h_ref, wpost_ref, bpost_ref, o_ref):
    acc = bpost_ref[...] + jnp.dot(
        h_ref[...], wpost_ref[H:, :], preferred_element_type=jnp.float32)
    for c in range(NCH):
        g = jnp.tanh(agg_ref[0, c] + agg_ref[1, c])
        acc += jnp.dot(g, wpost_ref[c * 128:(c + 1) * 128, :],
                       preferred_element_type=jnp.float32)
    o_ref[...] = jnp.maximum(acc, 0.0) + jnp.log1p(jnp.exp(-jnp.abs(acc)))


def _post(agg, h, W_post, b_post):
    grid = (N // BN,)
    return pl.pallas_call(
        _post_body,
        grid=grid,
        in_specs=[
            pl.BlockSpec((2, NCH, BN, 128), lambda i: (0, 0, i, 0)),
            pl.BlockSpec((BN, H), lambda i: (i, 0)),
            pl.BlockSpec((2 * H, D), lambda i: (0, 0)),
            pl.BlockSpec((1, D), lambda i: (0, 0)),
        ],
        out_specs=pl.BlockSpec((BN, D), lambda i: (i, 0)),
        out_shape=jax.ShapeDtypeStruct((N, D), jnp.float32),
    )(agg, h, W_post, b_post.reshape(1, D))


def kernel(x, edge_index, W_pre, b_pre, W_mp, b_mp, W_post, b_post):
    src = edge_index[0].astype(jnp.int32)
    dst = edge_index[1].astype(jnp.int32)
    e = src.shape[0]
    nb = NB
    pad = NW * EB * nb - e
    src_r = jnp.concatenate([src, jnp.zeros((pad,), jnp.int32)]).reshape(NW, nb, EB)
    dst_r = jnp.concatenate([dst, jnp.full((pad,), N, jnp.int32)]).reshape(NW, nb, EB)

    h, *zs = _pre(x, W_pre, b_pre, W_mp, b_mp)
    agg = _segsum(src_r, dst_r, zs)
    return _post(agg, h, W_post, b_post)


# speedup vs baseline: 3.0717x; 1.2954x over previous
import jax
import jax.numpy as jnp
from jax import lax
from jax.experimental import pallas as pl
from jax.experimental.pallas import tpu as pltpu
from jax.experimental.pallas import tpu_sc as plsc

N = 10000
D = 128
H = 1024
NCH = H // 128
BN = 400
NW = 32
EB = 256
R_MAIN = 632
E_TOTAL = 320000
NB = -(-E_TOTAL // (NW * EB))
_EL = E_TOTAL - (NW - 1) * NB * EB
NB_LAST = -(-_EL // EB)
R_LAST = N - 15 * R_MAIN
AGG_ROWS = N + 8


def _pre_body(x_ref, wpre_ref, bpre_ref, wmp_ref, bmp_ref, h_ref, *z_refs):
    xb = x_ref[...]
    h = jnp.tanh(jnp.dot(xb, wpre_ref[...], preferred_element_type=jnp.float32)
                 + bpre_ref[...])
    z = jnp.dot(h, wmp_ref[...], preferred_element_type=jnp.float32) + bmp_ref[...]
    h_ref[...] = h
    for c in range(NCH):
        z_refs[c][...] = z[:, c * 128:(c + 1) * 128]


def _pre(x, W_pre, b_pre, W_mp, b_mp):
    grid = (N // BN,)
    return pl.pallas_call(
        _pre_body,
        grid=grid,
        in_specs=[
            pl.BlockSpec((BN, D), lambda i: (i, 0)),
            pl.BlockSpec((D, H), lambda i: (0, 0)),
            pl.BlockSpec((1, H), lambda i: (0, 0)),
            pl.BlockSpec((H, H), lambda i: (0, 0)),
            pl.BlockSpec((1, H), lambda i: (0, 0)),
        ],
        out_specs=[pl.BlockSpec((BN, H), lambda i: (i, 0))]
        + [pl.BlockSpec((BN, 128), lambda i: (i, 0)) for _ in range(NCH)],
        out_shape=[jax.ShapeDtypeStruct((N, H), jnp.float32)]
        + [jax.ShapeDtypeStruct((N, 128), jnp.float32) for _ in range(NCH)],
    )(x, W_pre, b_pre.reshape(1, H), W_mp, b_mp.reshape(1, H))


def _segsum_body(src_hbm, dst_hbm, *rest):
    z_hbms = rest[:NCH]
    out_hbm = rest[NCH]
    sidx_all, didx_all, rows_a, agg_sh, sem_a = rest[NCH + 1:]

    core = lax.axis_index("c")
    sub = lax.axis_index("s")
    wid = core * 16 + sub
    r0 = sub * R_MAIN
    last = sub == 15
    zero16 = jnp.zeros((16,), jnp.float32)

    def _zero_rows_a():
        def _zrow(r, _):
            for k in range(128 // 16):
                rows_a[r, pl.ds(k * 16, 16)] = zero16
            return _

        lax.fori_loop(0, EB, _zrow, None)

    def _zero_span(base, total):
        for off in range(0, total, EB):
            n = min(EB, total - off)
            pltpu.sync_copy(rows_a.at[pl.ds(0, n)],
                            agg_sh.at[pl.ds(base + off, n)])

    def _zero_my_slice():
        _zero_rows_a()

        @pl.when(jnp.logical_not(last))
        def _():
            _zero_span(r0, R_MAIN)

        @pl.when(last)
        def _():
            _zero_span(r0, R_LAST + 8)

    _zero_my_slice()
    plsc.subcore_barrier()

    nbw = jnp.where(wid == NW - 1, NB_LAST, NB)
    for c in range(NCH):
        z_hbm = z_hbms[c]

        def _batch(b, _):
            pltpu.sync_copy(src_hbm.at[wid, b], sidx_all)
            pltpu.sync_copy(dst_hbm.at[wid, b], didx_all)
            pltpu.async_copy(z_hbm.at[sidx_all], rows_a, sem_a).wait()
            pltpu.sync_copy(rows_a, agg_sh.at[didx_all], add=True)
            return _

        lax.fori_loop(0, nbw, _batch, None)
        plsc.subcore_barrier()

        @pl.when(jnp.logical_not(last))
        def _():
            pltpu.sync_copy(agg_sh.at[pl.ds(r0, R_MAIN)],
                            out_hbm.at[core, c, pl.ds(r0, R_MAIN)])

        @pl.when(last)
        def _():
            pltpu.sync_copy(agg_sh.at[pl.ds(r0, R_LAST)],
                            out_hbm.at[core, c, pl.ds(r0, R_LAST)])

        if c + 1 < NCH:
            _zero_my_slice()
            plsc.subcore_barrier()


def _segsum(src_r, dst_r, zs):
    mesh = plsc.VectorSubcoreMesh(core_axis_name="c", subcore_axis_name="s")
    f = pl.kernel(
        _segsum_body,
        mesh=mesh,
        out_type=jax.ShapeDtypeStruct((2, NCH, N, 128), jnp.float32),
        scratch_types=[
            pltpu.VMEM((EB,), jnp.int32),
            pltpu.VMEM((EB,), jnp.int32),
            pltpu.VMEM((EB, 128), jnp.float32),
            pltpu.VMEM_SHARED((AGG_ROWS, 128), jnp.float32),
            pltpu.SemaphoreType.DMA,
        ],
    )
    return f(src_r, dst_r, *zs)


def _post_body(agg_ref, h_ref, wpost_ref, bpost_ref, o_ref):
    acc = bpost_ref[...] + jnp.dot(
        h_ref[...], wpost_ref[H:, :], preferred_element_type=jnp.float32)
    for c in range(NCH):
        g = jnp.tanh(agg_ref[0, c] + agg_ref[1, c])
        acc += jnp.dot(g, wpost_ref[c * 128:(c + 1) * 128, :],
                       preferred_element_type=jnp.float32)
    o_ref[...] = jnp.maximum(acc, 0.0) + jnp.log1p(jnp.exp(-jnp.abs(acc)))


def _post(agg, h, W_post, b_post):
    grid = (N // BN,)
    return pl.pallas_call(
        _post_body,
        grid=grid,
        in_specs=[
            pl.BlockSpec((2, NCH, BN, 128), lambda i: (0, 0, i, 0)),
            pl.BlockSpec((BN, H), lambda i: (i, 0)),
            pl.BlockSpec((2 * H, D), lambda i: (0, 0)),
            pl.BlockSpec((1, D), lambda i: (0, 0)),
        ],
        out_specs=pl.BlockSpec((BN, D), lambda i: (i, 0)),
        out_shape=jax.ShapeDtypeStruct((N, D), jnp.float32),
    )(agg, h, W_post, b_post.reshape(1, D))


def kernel(x, edge_index, W_pre, b_pre, W_mp, b_mp, W_post, b_post):
    src = edge_index[0].astype(jnp.int32)
    dst = edge_index[1].astype(jnp.int32)
    e = src.shape[0]
    nb = NB
    pad = NW * EB * nb - e
    src_r = jnp.concatenate([src, jnp.zeros((pad,), jnp.int32)]).reshape(NW, nb, EB)
    dst_r = jnp.concatenate([dst, jnp.full((pad,), N, jnp.int32)]).reshape(NW, nb, EB)

    h, *zs = _pre(x, W_pre, b_pre, W_mp, b_mp)
    agg = _segsum(src_r, dst_r, zs)
    return _post(agg, h, W_post, b_post)


# speedup vs baseline: 3.2462x; 1.0568x over previous
import jax
import jax.numpy as jnp
from jax import lax
from jax.experimental import pallas as pl
from jax.experimental.pallas import tpu as pltpu
from jax.experimental.pallas import tpu_sc as plsc

N = 10000
D = 128
H = 1024
NCH = H // 128
BN = 400
NW = 32
EB = 320
R_MAIN = 632
E_TOTAL = 320000
NB = -(-E_TOTAL // (NW * EB))
_EL = E_TOTAL - (NW - 1) * NB * EB
NB_LAST = -(-_EL // EB)
R_LAST = N - 15 * R_MAIN
AGG_ROWS = N + 8


def _pre_body(x_ref, wpre_ref, bpre_ref, wmp_ref, bmp_ref, h_ref, *z_refs):
    xb = x_ref[...]
    h = jnp.tanh(jnp.dot(xb, wpre_ref[...], preferred_element_type=jnp.float32)
                 + bpre_ref[...])
    z = jnp.dot(h, wmp_ref[...], preferred_element_type=jnp.float32) + bmp_ref[...]
    h_ref[...] = h
    for c in range(NCH):
        z_refs[c][...] = z[:, c * 128:(c + 1) * 128]


def _pre(x, W_pre, b_pre, W_mp, b_mp):
    grid = (N // BN,)
    return pl.pallas_call(
        _pre_body,
        grid=grid,
        in_specs=[
            pl.BlockSpec((BN, D), lambda i: (i, 0)),
            pl.BlockSpec((D, H), lambda i: (0, 0)),
            pl.BlockSpec((1, H), lambda i: (0, 0)),
            pl.BlockSpec((H, H), lambda i: (0, 0)),
            pl.BlockSpec((1, H), lambda i: (0, 0)),
        ],
        out_specs=[pl.BlockSpec((BN, H), lambda i: (i, 0))]
        + [pl.BlockSpec((BN, 128), lambda i: (i, 0)) for _ in range(NCH)],
        out_shape=[jax.ShapeDtypeStruct((N, H), jnp.float32)]
        + [jax.ShapeDtypeStruct((N, 128), jnp.float32) for _ in range(NCH)],
    )(x, W_pre, b_pre.reshape(1, H), W_mp, b_mp.reshape(1, H))


def _segsum_body(src_hbm, dst_hbm, *rest):
    z_hbms = rest[:NCH]
    out_hbm = rest[NCH]
    sidx_all, didx_all, rows_a, agg_sh, sem_a = rest[NCH + 1:]

    core = lax.axis_index("c")
    sub = lax.axis_index("s")
    wid = core * 16 + sub
    r0 = sub * R_MAIN
    last = sub == 15
    zero16 = jnp.zeros((16,), jnp.float32)

    def _zero_rows_a():
        def _zrow(r, _):
            for k in range(128 // 16):
                rows_a[r, pl.ds(k * 16, 16)] = zero16
            return _

        lax.fori_loop(0, EB, _zrow, None)

    def _zero_span(base, total):
        for off in range(0, total, EB):
            n = min(EB, total - off)
            pltpu.sync_copy(rows_a.at[pl.ds(0, n)],
                            agg_sh.at[pl.ds(base + off, n)])

    def _zero_my_slice():
        _zero_rows_a()

        @pl.when(jnp.logical_not(last))
        def _():
            _zero_span(r0, R_MAIN)

        @pl.when(last)
        def _():
            _zero_span(r0, R_LAST + 8)

    _zero_my_slice()
    plsc.subcore_barrier()

    nbw = jnp.where(wid == NW - 1, NB_LAST, NB)
    for c in range(NCH):
        z_hbm = z_hbms[c]

        def _batch(b, _):
            pltpu.sync_copy(src_hbm.at[wid, b], sidx_all)
            pltpu.sync_copy(dst_hbm.at[wid, b], didx_all)
            pltpu.async_copy(z_hbm.at[sidx_all], rows_a, sem_a).wait()
            pltpu.sync_copy(rows_a, agg_sh.at[didx_all], add=True)
            return _

        lax.fori_loop(0, nbw, _batch, None)
        plsc.subcore_barrier()

        @pl.when(jnp.logical_not(last))
        def _():
            pltpu.sync_copy(agg_sh.at[pl.ds(r0, R_MAIN)],
                            out_hbm.at[core, c, pl.ds(r0, R_MAIN)])

        @pl.when(last)
        def _():
            pltpu.sync_copy(agg_sh.at[pl.ds(r0, R_LAST)],
                            out_hbm.at[core, c, pl.ds(r0, R_LAST)])

        if c + 1 < NCH:
            _zero_my_slice()
            plsc.subcore_barrier()


def _segsum(src_r, dst_r, zs):
    mesh = plsc.VectorSubcoreMesh(core_axis_name="c", subcore_axis_name="s")
    f = pl.kernel(
        _segsum_body,
        mesh=mesh,
        out_type=jax.ShapeDtypeStruct((2, NCH, N, 128), jnp.float32),
        scratch_types=[
            pltpu.VMEM((EB,), jnp.int32),
            pltpu.VMEM((EB,), jnp.int32),
            pltpu.VMEM((EB, 128), jnp.float32),
            pltpu.VMEM_SHARED((AGG_ROWS, 128), jnp.float32),
            pltpu.SemaphoreType.DMA,
        ],
    )
    return f(src_r, dst_r, *zs)


def _post_body(agg_ref, h_ref, wpost_ref, bpost_ref, o_ref):
    acc = bpost_ref[...] + jnp.dot(
        h_ref[...], wpost_ref[H:, :], preferred_element_type=jnp.float32)
    for c in range(NCH):
        g = jnp.tanh(agg_ref[0, c] + agg_ref[1, c])
        acc += jnp.dot(g, wpost_ref[c * 128:(c + 1) * 128, :],
                       preferred_element_type=jnp.float32)
    o_ref[...] = jnp.maximum(acc, 0.0) + jnp.log1p(jnp.exp(-jnp.abs(acc)))


def _post(agg, h, W_post, b_post):
    grid = (N // BN,)
    return pl.pallas_call(
        _post_body,
        grid=grid,
        in_specs=[
            pl.BlockSpec((2, NCH, BN, 128), lambda i: (0, 0, i, 0)),
            pl.BlockSpec((BN, H), lambda i: (i, 0)),
            pl.BlockSpec((2 * H, D), lambda i: (0, 0)),
            pl.BlockSpec((1, D), lambda i: (0, 0)),
        ],
        out_specs=pl.BlockSpec((BN, D), lambda i: (i, 0)),
        out_shape=jax.ShapeDtypeStruct((N, D), jnp.float32),
    )(agg, h, W_post, b_post.reshape(1, D))


def kernel(x, edge_index, W_pre, b_pre, W_mp, b_mp, W_post, b_post):
    src = edge_index[0].astype(jnp.int32)
    dst = edge_index[1].astype(jnp.int32)
    e = src.shape[0]
    nb = NB
    pad = NW * EB * nb - e
    src_r = jnp.concatenate([src, jnp.zeros((pad,), jnp.int32)]).reshape(NW, nb, EB)
    dst_r = jnp.concatenate([dst, jnp.full((pad,), N, jnp.int32)]).reshape(NW, nb, EB)

    h, *zs = _pre(x, W_pre, b_pre, W_mp, b_mp)
    agg = _segsum(src_r, dst_r, zs)
    return _post(agg, h, W_post, b_post)


# speedup vs baseline: 3.3357x; 1.0276x over previous
import jax
import jax.numpy as jnp
from jax import lax
from jax.experimental import pallas as pl
from jax.experimental.pallas import tpu as pltpu
from jax.experimental.pallas import tpu_sc as plsc

N = 10000
D = 128
H = 1024
NCH = H // 128
BN = 400
NW = 32
EB = 384
R_MAIN = 632
E_TOTAL = 320000
NB = -(-E_TOTAL // (NW * EB))
R_LAST = N - 15 * R_MAIN
AGG_ROWS = N + 8


def _pre_body(x_ref, wpre_ref, bpre_ref, wmp_ref, bmp_ref, h_ref, *z_refs):
    xb = x_ref[...]
    h = jnp.tanh(jnp.dot(xb, wpre_ref[...], preferred_element_type=jnp.float32)
                 + bpre_ref[...])
    z = jnp.dot(h, wmp_ref[...], preferred_element_type=jnp.float32) + bmp_ref[...]
    h_ref[...] = h
    for c in range(NCH):
        z_refs[c][...] = z[:, c * 128:(c + 1) * 128]


def _pre(x, W_pre, b_pre, W_mp, b_mp):
    grid = (N // BN,)
    return pl.pallas_call(
        _pre_body,
        grid=grid,
        in_specs=[
            pl.BlockSpec((BN, D), lambda i: (i, 0)),
            pl.BlockSpec((D, H), lambda i: (0, 0)),
            pl.BlockSpec((1, H), lambda i: (0, 0)),
            pl.BlockSpec((H, H), lambda i: (0, 0)),
            pl.BlockSpec((1, H), lambda i: (0, 0)),
        ],
        out_specs=[pl.BlockSpec((BN, H), lambda i: (i, 0))]
        + [pl.BlockSpec((BN, 128), lambda i: (i, 0)) for _ in range(NCH)],
        out_shape=[jax.ShapeDtypeStruct((N, H), jnp.float32)]
        + [jax.ShapeDtypeStruct((N, 128), jnp.float32) for _ in range(NCH)],
    )(x, W_pre, b_pre.reshape(1, H), W_mp, b_mp.reshape(1, H))


def _segsum_body(src_hbm, dst_hbm, *rest):
    z_hbms = rest[:NCH]
    out_hbm = rest[NCH]
    sidx_all, didx_all, rows_a, agg_sh, sem_a = rest[NCH + 1:]

    core = lax.axis_index("c")
    sub = lax.axis_index("s")
    wid = core * 16 + sub
    r0 = sub * R_MAIN
    last = sub == 15
    zero16 = jnp.zeros((16,), jnp.float32)

    def _zero_rows_a():
        def _zrow(r, _):
            for k in range(128 // 16):
                rows_a[r, pl.ds(k * 16, 16)] = zero16
            return _

        lax.fori_loop(0, EB, _zrow, None)

    def _zero_span(base, total):
        for off in range(0, total, EB):
            n = min(EB, total - off)
            pltpu.sync_copy(rows_a.at[pl.ds(0, n)],
                            agg_sh.at[pl.ds(base + off, n)])

    def _zero_my_slice():
        _zero_rows_a()

        @pl.when(jnp.logical_not(last))
        def _():
            _zero_span(r0, R_MAIN)

        @pl.when(last)
        def _():
            _zero_span(r0, R_LAST + 8)

    _zero_my_slice()
    plsc.subcore_barrier()

    nbw = jnp.clip((E_TOTAL - wid * NB * EB + EB - 1) // EB, 0, NB)
    for c in range(NCH):
        z_hbm = z_hbms[c]

        def _batch(b, _):
            pltpu.sync_copy(src_hbm.at[wid, b], sidx_all)
            pltpu.sync_copy(dst_hbm.at[wid, b], didx_all)
            pltpu.async_copy(z_hbm.at[sidx_all], rows_a, sem_a).wait()
            pltpu.sync_copy(rows_a, agg_sh.at[didx_all], add=True)
            return _

        lax.fori_loop(0, nbw, _batch, None)
        plsc.subcore_barrier()

        @pl.when(jnp.logical_not(last))
        def _():
            pltpu.sync_copy(agg_sh.at[pl.ds(r0, R_MAIN)],
                            out_hbm.at[core, c, pl.ds(r0, R_MAIN)])

        @pl.when(last)
        def _():
            pltpu.sync_copy(agg_sh.at[pl.ds(r0, R_LAST)],
                            out_hbm.at[core, c, pl.ds(r0, R_LAST)])

        if c + 1 < NCH:
            _zero_my_slice()
            plsc.subcore_barrier()


def _segsum(src_r, dst_r, zs):
    mesh = plsc.VectorSubcoreMesh(core_axis_name="c", subcore_axis_name="s")
    f = pl.kernel(
        _segsum_body,
        mesh=mesh,
        out_type=jax.ShapeDtypeStruct((2, NCH, N, 128), jnp.float32),
        scratch_types=[
            pltpu.VMEM((EB,), jnp.int32),
            pltpu.VMEM((EB,), jnp.int32),
            pltpu.VMEM((EB, 128), jnp.float32),
            pltpu.VMEM_SHARED((AGG_ROWS, 128), jnp.float32),
            pltpu.SemaphoreType.DMA,
        ],
    )
    return f(src_r, dst_r, *zs)


def _post_body(agg_ref, h_ref, wpost_ref, bpost_ref, o_ref):
    acc = bpost_ref[...] + jnp.dot(
        h_ref[...], wpost_ref[H:, :], preferred_element_type=jnp.float32)
    for c in range(NCH):
        g = jnp.tanh(agg_ref[0, c] + agg_ref[1, c])
        acc += jnp.dot(g, wpost_ref[c * 128:(c + 1) * 128, :],
                       preferred_element_type=jnp.float32)
    o_ref[...] = jnp.maximum(acc, 0.0) + jnp.log1p(jnp.exp(-jnp.abs(acc)))


def _post(agg, h, W_post, b_post):
    grid = (N // BN,)
    return pl.pallas_call(
        _post_body,
        grid=grid,
        in_specs=[
            pl.BlockSpec((2, NCH, BN, 128), lambda i: (0, 0, i, 0)),
            pl.BlockSpec((BN, H), lambda i: (i, 0)),
            pl.BlockSpec((2 * H, D), lambda i: (0, 0)),
            pl.BlockSpec((1, D), lambda i: (0, 0)),
        ],
        out_specs=pl.BlockSpec((BN, D), lambda i: (i, 0)),
        out_shape=jax.ShapeDtypeStruct((N, D), jnp.float32),
    )(agg, h, W_post, b_post.reshape(1, D))


def kernel(x, edge_index, W_pre, b_pre, W_mp, b_mp, W_post, b_post):
    src = edge_index[0].astype(jnp.int32)
    dst = edge_index[1].astype(jnp.int32)
    e = src.shape[0]
    nb = NB
    pad = NW * EB * nb - e
    src_r = jnp.concatenate([src, jnp.zeros((pad,), jnp.int32)]).reshape(NW, nb, EB)
    dst_r = jnp.concatenate([dst, jnp.full((pad,), N, jnp.int32)]).reshape(NW, nb, EB)

    h, *zs = _pre(x, W_pre, b_pre, W_mp, b_mp)
    agg = _segsum(src_r, dst_r, zs)
    return _post(agg, h, W_post, b_post)


# speedup vs baseline: 4.2751x; 1.2816x over previous
import jax
import jax.numpy as jnp
from jax import lax
from jax.experimental import pallas as pl
from jax.experimental.pallas import tpu as pltpu
from jax.experimental.pallas import tpu_sc as plsc

N = 10000
D = 128
H = 1024
NCH = H // 128
BN = 400
NW = 32
EB = 192
R_MAIN = 632
E_TOTAL = 320000
NB = -(-E_TOTAL // (NW * EB))
R_LAST = N - 15 * R_MAIN
AGG_ROWS = N + 8


def _pre_body(x_ref, wpre_ref, bpre_ref, wmp_ref, bmp_ref, h_ref, *z_refs):
    xb = x_ref[...]
    h = jnp.tanh(jnp.dot(xb, wpre_ref[...], preferred_element_type=jnp.float32)
                 + bpre_ref[...])
    z = jnp.dot(h, wmp_ref[...], preferred_element_type=jnp.float32) + bmp_ref[...]
    h_ref[...] = h
    for c in range(NCH):
        z_refs[c][...] = z[:, c * 128:(c + 1) * 128]


def _pre(x, W_pre, b_pre, W_mp, b_mp):
    grid = (N // BN,)
    return pl.pallas_call(
        _pre_body,
        grid=grid,
        in_specs=[
            pl.BlockSpec((BN, D), lambda i: (i, 0)),
            pl.BlockSpec((D, H), lambda i: (0, 0)),
            pl.BlockSpec((1, H), lambda i: (0, 0)),
            pl.BlockSpec((H, H), lambda i: (0, 0)),
            pl.BlockSpec((1, H), lambda i: (0, 0)),
        ],
        out_specs=[pl.BlockSpec((BN, H), lambda i: (i, 0))]
        + [pl.BlockSpec((BN, 128), lambda i: (i, 0)) for _ in range(NCH)],
        out_shape=[jax.ShapeDtypeStruct((N, H), jnp.float32)]
        + [jax.ShapeDtypeStruct((N, 128), jnp.float32) for _ in range(NCH)],
    )(x, W_pre, b_pre.reshape(1, H), W_mp, b_mp.reshape(1, H))


def _segsum_body(src_hbm, dst_hbm, *rest):
    z_hbms = rest[:NCH]
    out_hbm = rest[NCH]
    (sidx_a, didx_a, sidx_b, didx_b, rows_a, rows_b,
     agg_sh, sem_a, sem_b) = rest[NCH + 1:]

    core = lax.axis_index("c")
    sub = lax.axis_index("s")
    wid = core * 16 + sub
    r0 = sub * R_MAIN
    last = sub == 15
    zero16 = jnp.zeros((16,), jnp.float32)

    def _zero_rows_a():
        def _zrow(r, _):
            for k in range(128 // 16):
                rows_a[r, pl.ds(k * 16, 16)] = zero16
            return _

        lax.fori_loop(0, EB, _zrow, None)

    def _zero_span(base, total):
        for off in range(0, total, EB):
            n = min(EB, total - off)
            pltpu.sync_copy(rows_a.at[pl.ds(0, n)],
                            agg_sh.at[pl.ds(base + off, n)])

    def _zero_my_slice():
        _zero_rows_a()

        @pl.when(jnp.logical_not(last))
        def _():
            _zero_span(r0, R_MAIN)

        @pl.when(last)
        def _():
            _zero_span(r0, R_LAST + 8)

    _zero_my_slice()
    plsc.subcore_barrier()

    nbw = jnp.clip((E_TOTAL - wid * NB * EB + EB - 1) // EB, 0, NB)
    npairs = nbw // 2
    for c in range(NCH):
        z_hbm = z_hbms[c]

        def _load(b, si, di):
            pltpu.sync_copy(src_hbm.at[wid, b], si)
            pltpu.sync_copy(dst_hbm.at[wid, b], di)

        def _start(si, rv, sem):
            pltpu.async_copy(z_hbm.at[si], rv, sem)

        def _finish(si, di, rv, sem):
            pltpu.make_async_copy(z_hbm.at[si], rv, sem).wait()
            pltpu.sync_copy(rv, agg_sh.at[di], add=True)

        @pl.when(nbw > 0)
        def _():
            _load(0, sidx_a, didx_a)
            _start(sidx_a, rows_a, sem_a)

        def _pair(p, _):
            b = 2 * p
            _load(b + 1, sidx_b, didx_b)
            _start(sidx_b, rows_b, sem_b)
            _finish(sidx_a, didx_a, rows_a, sem_a)

            @pl.when(b + 2 < nbw)
            def _():
                _load(b + 2, sidx_a, didx_a)
                _start(sidx_a, rows_a, sem_a)

            _finish(sidx_b, didx_b, rows_b, sem_b)
            return _

        lax.fori_loop(0, npairs, _pair, None)

        @pl.when(nbw % 2 == 1)
        def _():
            _finish(sidx_a, didx_a, rows_a, sem_a)

        plsc.subcore_barrier()

        @pl.when(jnp.logical_not(last))
        def _():
            pltpu.sync_copy(agg_sh.at[pl.ds(r0, R_MAIN)],
                            out_hbm.at[core, c, pl.ds(r0, R_MAIN)])

        @pl.when(last)
        def _():
            pltpu.sync_copy(agg_sh.at[pl.ds(r0, R_LAST)],
                            out_hbm.at[core, c, pl.ds(r0, R_LAST)])

        if c + 1 < NCH:
            _zero_my_slice()
            plsc.subcore_barrier()


def _segsum(src_r, dst_r, zs):
    mesh = plsc.VectorSubcoreMesh(core_axis_name="c", subcore_axis_name="s")
    f = pl.kernel(
        _segsum_body,
        mesh=mesh,
        out_type=jax.ShapeDtypeStruct((2, NCH, N, 128), jnp.float32),
        scratch_types=[
            pltpu.VMEM((EB,), jnp.int32),
            pltpu.VMEM((EB,), jnp.int32),
            pltpu.VMEM((EB,), jnp.int32),
            pltpu.VMEM((EB,), jnp.int32),
            pltpu.VMEM((EB, 128), jnp.float32),
            pltpu.VMEM((EB, 128), jnp.float32),
            pltpu.VMEM_SHARED((AGG_ROWS, 128), jnp.float32),
            pltpu.SemaphoreType.DMA,
            pltpu.SemaphoreType.DMA,
        ],
    )
    return f(src_r, dst_r, *zs)


def _post_body(agg_ref, h_ref, wpost_ref, bpost_ref, o_ref):
    acc = bpost_ref[...] + jnp.dot(
        h_ref[...], wpost_ref[H:, :], preferred_element_type=jnp.float32)
    for c in range(NCH):
        g = jnp.tanh(agg_ref[0, c] + agg_ref[1, c])
        acc += jnp.dot(g, wpost_ref[c * 128:(c + 1) * 128, :],
                       preferred_element_type=jnp.float32)
    o_ref[...] = jnp.maximum(acc, 0.0) + jnp.log1p(jnp.exp(-jnp.abs(acc)))


def _post(agg, h, W_post, b_post):
    grid = (N // BN,)
    return pl.pallas_call(
        _post_body,
        grid=grid,
        in_specs=[
            pl.BlockSpec((2, NCH, BN, 128), lambda i: (0, 0, i, 0)),
            pl.BlockSpec((BN, H), lambda i: (i, 0)),
            pl.BlockSpec((2 * H, D), lambda i: (0, 0)),
            pl.BlockSpec((1, D), lambda i: (0, 0)),
        ],
        out_specs=pl.BlockSpec((BN, D), lambda i: (i, 0)),
        out_shape=jax.ShapeDtypeStruct((N, D), jnp.float32),
    )(agg, h, W_post, b_post.reshape(1, D))


def kernel(x, edge_index, W_pre, b_pre, W_mp, b_mp, W_post, b_post):
    src = edge_index[0].astype(jnp.int32)
    dst = edge_index[1].astype(jnp.int32)
    e = src.shape[0]
    nb = NB
    pad = NW * EB * nb - e
    src_r = jnp.concatenate([src, jnp.zeros((pad,), jnp.int32)]).reshape(NW, nb, EB)
    dst_r = jnp.concatenate([dst, jnp.full((pad,), N, jnp.int32)]).reshape(NW, nb, EB)

    h, *zs = _pre(x, W_pre, b_pre, W_mp, b_mp)
    agg = _segsum(src_r, dst_r, zs)
    return _post(agg, h, W_post, b_post)
